# baseline (speedup 1.0000x reference)
import functools

import jax
import jax.numpy as jnp
from jax import lax
from jax.experimental import pallas as pl
from jax.experimental.pallas import tpu as pltpu
from jax.experimental.pallas import tpu_sc as plsc

N_DAYS = 365
N_STEPS = 48
N_IDS = 1024
BATCH = 1024
ROW = N_STEPS * N_IDS

NC = 2
NS = 16
NW = NC * NS
BPW = BATCH // NW


def _decode_body(cos_ref, sin_ref, idx_ref):
    two_pi = 2.0 * jnp.pi
    ang = jnp.arctan2(sin_ref[...], cos_ref[...])
    doy = jnp.round(jnp.mod(ang, two_pi) / two_pi * 365.0)
    idx_ref[...] = doy.astype(jnp.int32) - 1


def _decode_idx(x2):
    m = x2.reshape(BATCH, 2)
    cos8 = m[:, 0].reshape(8, BATCH // 8)
    sin8 = m[:, 1].reshape(8, BATCH // 8)
    idx8 = pl.pallas_call(
        _decode_body,
        out_shape=jax.ShapeDtypeStruct((8, BATCH // 8), jnp.int32),
    )(cos8, sin8)
    return idx8.reshape(BATCH)


def _gather_body(lut_hbm, idx_hbm, out_hbm, idx_v, sem):
    wid = lax.axis_index("s") * NC + lax.axis_index("c")
    base = wid * BPW
    pltpu.sync_copy(idx_hbm.at[pl.ds(base, BPW)], idx_v)

    copies = []
    for b in range(BPW):
        if b % 16 == 0:
            chunk = idx_v[pl.ds(b, 16)]
        d = chunk[b % 16]
        copies.append(pltpu.async_copy(
            lut_hbm.at[pl.ds(d, 1)],
            out_hbm.at[pl.ds(base + b, 1)], sem))
    for c in copies:
        c.wait()


_sc_gather = functools.partial(
    pl.kernel,
    out_type=jax.ShapeDtypeStruct((BATCH, ROW), jnp.float32),
    mesh=plsc.VectorSubcoreMesh(core_axis_name="c", subcore_axis_name="s",
                                num_cores=NC, num_subcores=NS),
    scratch_types=[
        pltpu.VMEM((BPW,), jnp.int32),
        pltpu.SemaphoreType.DMA,
    ],
)(_gather_body)


def kernel(x1, x2, lut):
    del x1
    idx = _decode_idx(x2)
    lut2 = lut.reshape(N_DAYS, ROW)
    out2 = _sc_gather(lut2, idx)
    return out2.reshape(BATCH, N_STEPS, N_IDS)


# speedup vs baseline: 15.4645x; 15.4645x over previous
import functools

import jax
import jax.numpy as jnp
from jax import lax
from jax.experimental import pallas as pl
from jax.experimental.pallas import tpu as pltpu
from jax.experimental.pallas import tpu_sc as plsc

N_DAYS = 365
N_STEPS = 48
N_IDS = 1024
BATCH = 1024
ROW = N_STEPS * N_IDS

NC = 2
NS = 16
NW = NC * NS
BPW = BATCH // NW


def _decode_body(cos_ref, sin_ref, idx_ref):
    two_pi = 2.0 * jnp.pi
    ang = jnp.arctan2(sin_ref[...], cos_ref[...])
    doy = jnp.round(jnp.mod(ang, two_pi) / two_pi * 365.0)
    idx_ref[...] = doy.astype(jnp.int32) - 1


def _decode_idx(x2):
    m = x2.reshape(BATCH, 2)
    cos8 = m[:, 0].reshape(8, BATCH // 8)
    sin8 = m[:, 1].reshape(8, BATCH // 8)
    idx8 = pl.pallas_call(
        _decode_body,
        out_shape=jax.ShapeDtypeStruct((8, BATCH // 8), jnp.int32),
    )(cos8, sin8)
    return idx8.reshape(BATCH)


def _gather_body(lut_hbm, idx_hbm, out_hbm,
                 idx_v, buf0, buf1, gsem0, gsem1, wsem0, wsem1):
    wid = lax.axis_index("s") * NC + lax.axis_index("c")
    base = wid * BPW
    pltpu.sync_copy(idx_hbm.at[pl.ds(base, BPW)], idx_v)

    bufs = (buf0, buf1)
    gsems = (gsem0, gsem1)
    wsems = (wsem0, wsem1)

    def start_gather(b):
        s = b % 2
        return pltpu.async_copy(
            lut_hbm.at[idx_v.at[b]], bufs[s], gsems[s])

    pend_w = [None, None]
    g = start_gather(0)
    for b in range(BPW):
        s = b % 2
        g_cur = g
        if b + 1 < BPW:
            if pend_w[(b + 1) % 2] is not None:
                pend_w[(b + 1) % 2].wait()
            g = start_gather(b + 1)
        g_cur.wait()
        pend_w[s] = pltpu.async_copy(
            bufs[s], out_hbm.at[pl.ds(base + b, 1)], wsems[s])
    for w in pend_w:
        if w is not None:
            w.wait()


_sc_gather = functools.partial(
    pl.kernel,
    out_type=jax.ShapeDtypeStruct((BATCH, ROW), jnp.float32),
    mesh=plsc.VectorSubcoreMesh(core_axis_name="c", subcore_axis_name="s",
                                num_cores=NC, num_subcores=NS),
    scratch_types=[
        pltpu.VMEM((BPW, 1), jnp.int32),
        pltpu.VMEM((1, ROW), jnp.float32),
        pltpu.VMEM((1, ROW), jnp.float32),
        pltpu.SemaphoreType.DMA,
        pltpu.SemaphoreType.DMA,
        pltpu.SemaphoreType.DMA,
        pltpu.SemaphoreType.DMA,
    ],
)(_gather_body)


def kernel(x1, x2, lut):
    del x1
    idx = _decode_idx(x2).reshape(BATCH, 1)
    lut2 = lut.reshape(N_DAYS, ROW)
    out2 = _sc_gather(lut2, idx)
    return out2.reshape(BATCH, N_STEPS, N_IDS)


# speedup vs baseline: 39.9710x; 2.5847x over previous
import functools

import jax
import jax.numpy as jnp
from jax import lax
from jax.experimental import pallas as pl
from jax.experimental.pallas import tpu as pltpu
from jax.experimental.pallas import tpu_sc as plsc

N_DAYS = 365
N_STEPS = 48
N_IDS = 1024
BATCH = 1024

NC = 2
NS = 16
NW = NC * NS
BPW = BATCH // NW


def _decode_body(cos_ref, sin_ref, idx_ref):
    two_pi = 2.0 * jnp.pi
    ang = jnp.arctan2(sin_ref[...], cos_ref[...])
    doy = jnp.round(jnp.mod(ang, two_pi) / two_pi * 365.0)
    idx_ref[...] = doy.astype(jnp.int32) - 1


def _decode_idx(x2):
    m = x2.reshape(BATCH, 2)
    cos8 = m[:, 0].reshape(8, BATCH // 8)
    sin8 = m[:, 1].reshape(8, BATCH // 8)
    idx8 = pl.pallas_call(
        _decode_body,
        out_shape=jax.ShapeDtypeStruct((8, BATCH // 8), jnp.int32),
    )(cos8, sin8)
    return idx8.reshape(BATCH)


def _gather_body(lut_hbm, idx_hbm, out_hbm,
                 idx_v, buf0, buf1, gsem0, gsem1, wsem0, wsem1):
    wid = lax.axis_index("s") * NC + lax.axis_index("c")
    base = wid * BPW
    pltpu.sync_copy(idx_hbm.at[pl.ds(base, BPW)], idx_v)

    bufs = (buf0, buf1)
    gsems = (gsem0, gsem1)
    wsems = (wsem0, wsem1)

    chunks = [idx_v[pl.ds(g * 16, 16)] for g in range(BPW // 16)]

    def start_gather(b):
        s = b % 2
        d = chunks[b // 16][b % 16]
        return pltpu.async_copy(
            lut_hbm.at[pl.ds(d, 1)], bufs[s], gsems[s])

    pend_w = [None, None]
    g = start_gather(0)
    for b in range(BPW):
        s = b % 2
        g_cur = g
        if b + 1 < BPW:
            if pend_w[(b + 1) % 2] is not None:
                pend_w[(b + 1) % 2].wait()
            g = start_gather(b + 1)
        g_cur.wait()
        pend_w[s] = pltpu.async_copy(
            bufs[s], out_hbm.at[pl.ds(base + b, 1)], wsems[s])
    for w in pend_w:
        if w is not None:
            w.wait()


_sc_gather = functools.partial(
    pl.kernel,
    out_type=jax.ShapeDtypeStruct((BATCH, N_STEPS, N_IDS), jnp.float32),
    mesh=plsc.VectorSubcoreMesh(core_axis_name="c", subcore_axis_name="s",
                                num_cores=NC, num_subcores=NS),
    scratch_types=[
        pltpu.VMEM((BPW,), jnp.int32),
        pltpu.VMEM((1, N_STEPS, N_IDS), jnp.float32),
        pltpu.VMEM((1, N_STEPS, N_IDS), jnp.float32),
        pltpu.SemaphoreType.DMA,
        pltpu.SemaphoreType.DMA,
        pltpu.SemaphoreType.DMA,
        pltpu.SemaphoreType.DMA,
    ],
    compiler_params=pltpu.CompilerParams(use_tc_tiling_on_sc=True),
)(_gather_body)


def kernel(x1, x2, lut):
    del x1
    idx = _decode_idx(x2)
    return _sc_gather(lut, idx)
